# baseline (speedup 1.0000x reference)
import functools

import numpy as np
import jax
import jax.numpy as jnp
from jax import lax
from jax.experimental import pallas as pl
from jax.experimental.pallas import tpu as pltpu
from jax.experimental.pallas import tpu_sc as plsc

_ROWS = 128
_VOCAB = 100000
_G = 16
_RPG = 8
_LANES = 16
_C = 3200
_NCH = 16
_HBASE = 48768
_TAIL = 99968
_TAILN = 32


def _rotl32(x, d):
    return (x << np.uint32(d)) | (x >> np.uint32(32 - d))


def _threefry2x32(k0, k1, x0, x1):
    ks0, ks1 = np.uint32(k0), np.uint32(k1)
    ks2 = ks0 ^ ks1 ^ np.uint32(0x1BD11BDA)
    rot = ((13, 15, 26, 6), (17, 29, 16, 24))
    ks = (ks0, ks1, ks2)
    x0 = x0 + ks0
    x1 = x1 + ks1
    for i in range(5):
        for r in rot[i % 2]:
            x0 = x0 + x1
            x1 = _rotl32(x1, r)
            x1 = x0 ^ x1
        x0 = x0 + ks[(i + 1) % 3]
        x1 = x1 + ks[(i + 2) % 3] + np.uint32(i + 1)
    return x0, x1


def _exponential_noise(shape, seed=42):
    n = int(np.prod(shape))
    idx = np.arange(n, dtype=np.uint64)
    c1 = (idx >> np.uint64(32)).astype(np.uint32)
    c2 = (idx & np.uint64(0xFFFFFFFF)).astype(np.uint32)
    b1, b2 = _threefry2x32(np.uint32(0), np.uint32(seed), c1, c2)
    bits = b1 ^ b2
    fb = (bits >> np.uint32(9)) | np.uint32(0x3F800000)
    u = fb.view(np.float32) - np.float32(1.0)
    return (-np.log1p(-u)).reshape(shape)


_NOISE = np.maximum(_exponential_noise((_ROWS, _VOCAB)), np.float32(1e-10))
_GUMBEL = (-np.log(_NOISE)).astype(np.float32)
_INV_NOISE = (np.float32(1.0) / _NOISE).astype(np.float32)


def _sc_body(logits_hbm, gumbel_hbm, tb_hbm,
             m1_hbm, i1_hbm, m2_hbm, i2_hbm,
             xbufs, gbufs, xtail, gtail, tbuf,
             sm1, si1, sm2, si2, sem0, sem1):
    wid = lax.axis_index("s") * 2 + lax.axis_index("c")
    grp = wid // 2
    half = wid - 2 * grp
    row0 = pl.multiple_of(grp * _RPG, 8)
    base = pl.multiple_of(half * _HBASE, 128)
    sems = (sem0, sem1)

    def issue(k, b):
        start = pl.multiple_of(base + k * _C, 128)
        pltpu.async_copy(
            logits_hbm.at[pl.ds(row0, _RPG), pl.ds(start, _C)], xbufs.at[b], sems[b])
        pltpu.async_copy(
            gumbel_hbm.at[pl.ds(row0, _RPG), pl.ds(start, _C)], gbufs.at[b], sems[b])

    def drain(b):
        pltpu.make_async_copy(
            logits_hbm.at[pl.ds(row0, _RPG), pl.ds(0, _C)], xbufs.at[b], sems[b]).wait()
        pltpu.make_async_copy(
            gumbel_hbm.at[pl.ds(row0, _RPG), pl.ds(0, _C)], gbufs.at[b], sems[b]).wait()

    pltpu.sync_copy(tb_hbm.at[pl.ds(row0, _RPG)], tbuf)

    lane = lax.iota(jnp.int32, _LANES)
    neg_inf = jnp.full((_LANES,), -jnp.inf, jnp.float32)
    zero_i = jnp.zeros((_LANES,), jnp.int32)
    for r in range(_RPG):
        for s in range(2):
            sm1[r, pl.ds(s * _LANES, _LANES)] = neg_inf
            si1[r, pl.ds(s * _LANES, _LANES)] = zero_i
            sm2[r, pl.ds(s * _LANES, _LANES)] = neg_inf
            si2[r, pl.ds(s * _LANES, _LANES)] = zero_i

    def scan_buf(xb, gb, nvec, chunk_base):
        for r in range(_RPG):
            t = tbuf[r, :]

            def body(i, c, xb=xb, gb=gb, t=t, chunk_base=chunk_base, r=r):
                out = []
                for s in range(2):
                    m1, i1, m2, i2 = c[4 * s:4 * s + 4]
                    off = i * (2 * _LANES) + s * _LANES
                    x = xb[r, pl.ds(off, _LANES)]
                    g = gb[r, pl.ds(off, _LANES)]
                    v = x + t * g
                    cur = lane + (chunk_base + off)
                    gt1 = v > m1
                    loser = jnp.minimum(v, m1)
                    loser_i = jnp.where(gt1, i1, cur)
                    gt2 = loser > m2
                    m2n = jnp.where(gt2, loser, m2)
                    i2n = jnp.where(gt2, loser_i, i2)
                    m1n = jnp.maximum(v, m1)
                    i1n = jnp.where(gt1, cur, i1)
                    out += [m1n, i1n, m2n, i2n]
                return tuple(out)

            carry = tuple(
                ref[r, pl.ds(s * _LANES, _LANES)]
                for s in range(2) for ref in (sm1, si1, sm2, si2))
            carry = lax.fori_loop(0, nvec // 2, body, carry, unroll=4)
            for s in range(2):
                sm1[r, pl.ds(s * _LANES, _LANES)] = carry[4 * s + 0]
                si1[r, pl.ds(s * _LANES, _LANES)] = carry[4 * s + 1]
                sm2[r, pl.ds(s * _LANES, _LANES)] = carry[4 * s + 2]
                si2[r, pl.ds(s * _LANES, _LANES)] = carry[4 * s + 3]

    issue(0, 0)
    issue(1, 1)

    def chunk_pair(p, _):
        k = p * 2
        for b in range(2):
            cur = k + b
            drain(b)
            scan_buf(xbufs.at[b], gbufs.at[b], _C // _LANES, base + cur * _C)

            @pl.when(cur + 2 < _NCH)
            def _(cur=cur, b=b):
                issue(cur + 2, b)
        return _

    lax.fori_loop(0, _NCH // 2, chunk_pair, None)

    pltpu.sync_copy(logits_hbm.at[pl.ds(row0, _RPG), pl.ds(_TAIL, _TAILN)], xtail)
    pltpu.sync_copy(gumbel_hbm.at[pl.ds(row0, _RPG), pl.ds(_TAIL, _TAILN)], gtail)
    scan_buf(xtail, gtail, _TAILN // _LANES, _TAIL)

    out0 = pl.multiple_of(half * _ROWS + grp * _RPG, 8)
    pltpu.sync_copy(sm1, m1_hbm.at[pl.ds(out0, _RPG)])
    pltpu.sync_copy(si1, i1_hbm.at[pl.ds(out0, _RPG)])
    pltpu.sync_copy(sm2, m2_hbm.at[pl.ds(out0, _RPG)])
    pltpu.sync_copy(si2, i2_hbm.at[pl.ds(out0, _RPG)])


_sc_top2 = functools.partial(
    pl.kernel,
    mesh=plsc.VectorSubcoreMesh(core_axis_name="c", subcore_axis_name="s"),
    out_type=[
        jax.ShapeDtypeStruct((2 * _ROWS, 2 * _LANES), jnp.float32),
        jax.ShapeDtypeStruct((2 * _ROWS, 2 * _LANES), jnp.int32),
        jax.ShapeDtypeStruct((2 * _ROWS, 2 * _LANES), jnp.float32),
        jax.ShapeDtypeStruct((2 * _ROWS, 2 * _LANES), jnp.int32),
    ],
    scratch_types=[
        pltpu.VMEM((2, _RPG, _C), jnp.float32),
        pltpu.VMEM((2, _RPG, _C), jnp.float32),
        pltpu.VMEM((_RPG, _TAILN), jnp.float32),
        pltpu.VMEM((_RPG, _TAILN), jnp.float32),
        pltpu.VMEM((_RPG, _LANES), jnp.float32),
        pltpu.VMEM((_RPG, 2 * _LANES), jnp.float32),
        pltpu.VMEM((_RPG, 2 * _LANES), jnp.int32),
        pltpu.VMEM((_RPG, 2 * _LANES), jnp.float32),
        pltpu.VMEM((_RPG, 2 * _LANES), jnp.int32),
        pltpu.SemaphoreType.DMA,
        pltpu.SemaphoreType.DMA,
    ],
)(_sc_body)


def kernel(logits, temperatures):
    logits = logits.astype(jnp.float32)
    temps = temperatures.astype(jnp.float32)
    gumbel = jnp.asarray(_GUMBEL)
    tb = jnp.broadcast_to(temps[:, None], (_ROWS, _LANES))
    m1, i1, m2, i2 = _sc_top2(logits, gumbel, tb)

    vals = jnp.concatenate(
        [m1[:_ROWS], m1[_ROWS:], m2[:_ROWS], m2[_ROWS:]], axis=1)
    idxs = jnp.concatenate(
        [i1[:_ROWS], i1[_ROWS:], i2[:_ROWS], i2[_ROWS:]], axis=1)
    a1 = jnp.argmax(vals, axis=1)
    c1 = jnp.take_along_axis(idxs, a1[:, None], axis=1)[:, 0]
    vals2 = jnp.where(idxs == c1[:, None], -jnp.inf, vals)
    a2 = jnp.argmax(vals2, axis=1)
    c2 = jnp.take_along_axis(idxs, a2[:, None], axis=1)[:, 0]
    cand = jnp.stack([c1, c2], axis=1)

    xg = jnp.take_along_axis(logits, cand, axis=1)
    ng = jnp.take_along_axis(jnp.asarray(_INV_NOISE), cand, axis=1)
    safe_t = jnp.maximum(temps[:, None], 1e-10)
    w = xg / safe_t
    r = jnp.exp(w - jnp.max(w, axis=1, keepdims=True)) * ng
    pick = jnp.argmax(r, axis=1)
    sample = jnp.take_along_axis(cand, pick[:, None], axis=1)[:, 0]

    g_hi = jnp.where(
        xg[:, 0] > xg[:, 1],
        cand[:, 0],
        jnp.where(xg[:, 1] > xg[:, 0], cand[:, 1],
                  jnp.minimum(cand[:, 0], cand[:, 1])),
    )
    return jnp.where(temps <= 1e-10, g_hi, sample)


# speedup vs baseline: 1.1121x; 1.1121x over previous
import functools

import numpy as np
import jax
import jax.numpy as jnp
from jax import lax
from jax.experimental import pallas as pl
from jax.experimental.pallas import tpu as pltpu
from jax.experimental.pallas import tpu_sc as plsc

_ROWS = 128
_VOCAB = 100000
_G = 16
_RPG = 8
_LANES = 16
_W = 66048
_C = 2304
_NCH = 8
_SC_BASE0 = 66048
_SC_HSTEP = 15488
_TAIL = 99968
_TAILN = 32


def _rotl32(x, d):
    return (x << np.uint32(d)) | (x >> np.uint32(32 - d))


def _threefry2x32(k0, k1, x0, x1):
    ks0, ks1 = np.uint32(k0), np.uint32(k1)
    ks2 = ks0 ^ ks1 ^ np.uint32(0x1BD11BDA)
    rot = ((13, 15, 26, 6), (17, 29, 16, 24))
    ks = (ks0, ks1, ks2)
    x0 = x0 + ks0
    x1 = x1 + ks1
    for i in range(5):
        for r in rot[i % 2]:
            x0 = x0 + x1
            x1 = _rotl32(x1, r)
            x1 = x0 ^ x1
        x0 = x0 + ks[(i + 1) % 3]
        x1 = x1 + ks[(i + 2) % 3] + np.uint32(i + 1)
    return x0, x1


def _exponential_noise(shape, seed=42):
    n = int(np.prod(shape))
    idx = np.arange(n, dtype=np.uint64)
    c1 = (idx >> np.uint64(32)).astype(np.uint32)
    c2 = (idx & np.uint64(0xFFFFFFFF)).astype(np.uint32)
    b1, b2 = _threefry2x32(np.uint32(0), np.uint32(seed), c1, c2)
    bits = b1 ^ b2
    fb = (bits >> np.uint32(9)) | np.uint32(0x3F800000)
    u = fb.view(np.float32) - np.float32(1.0)
    return (-np.log1p(-u)).reshape(shape)


_NOISE = np.maximum(_exponential_noise((_ROWS, _VOCAB)), np.float32(1e-10))
_GUMBEL = (-np.log(_NOISE)).astype(np.float32)
_INV_NOISE = (np.float32(1.0) / _NOISE).astype(np.float32)


def _sc_body(logits_hbm, gumbel_hbm, tb_hbm,
             m1_hbm, i1_hbm, m2_hbm, i2_hbm,
             xbufs, gbufs, xtail, gtail, tbuf,
             sm1, si1, sm2, si2, sem0, sem1):
    wid = lax.axis_index("s") * 2 + lax.axis_index("c")
    grp = wid // 2
    half = wid - 2 * grp
    row0 = pl.multiple_of(grp * _RPG, 8)
    base = pl.multiple_of(_SC_BASE0 + half * _SC_HSTEP, 128)
    sems = (sem0, sem1)

    def issue(k, b):
        start = pl.multiple_of(base + k * _C, 128)
        pltpu.async_copy(
            logits_hbm.at[pl.ds(row0, _RPG), pl.ds(start, _C)], xbufs.at[b], sems[b])
        pltpu.async_copy(
            gumbel_hbm.at[pl.ds(row0, _RPG), pl.ds(start, _C)], gbufs.at[b], sems[b])

    def drain(b):
        pltpu.make_async_copy(
            logits_hbm.at[pl.ds(row0, _RPG), pl.ds(0, _C)], xbufs.at[b], sems[b]).wait()
        pltpu.make_async_copy(
            gumbel_hbm.at[pl.ds(row0, _RPG), pl.ds(0, _C)], gbufs.at[b], sems[b]).wait()

    pltpu.sync_copy(tb_hbm.at[pl.ds(row0, _RPG)], tbuf)

    lane = lax.iota(jnp.int32, _LANES)
    neg_inf = jnp.full((_LANES,), -jnp.inf, jnp.float32)
    zero_i = jnp.zeros((_LANES,), jnp.int32)
    for r in range(_RPG):
        for s in range(2):
            sm1[r, pl.ds(s * _LANES, _LANES)] = neg_inf
            si1[r, pl.ds(s * _LANES, _LANES)] = zero_i
            sm2[r, pl.ds(s * _LANES, _LANES)] = neg_inf
            si2[r, pl.ds(s * _LANES, _LANES)] = zero_i

    def scan_buf(xb, gb, nvec, chunk_base):
        for r in range(_RPG):
            t = tbuf[r, :]

            def body(i, c, xb=xb, gb=gb, t=t, chunk_base=chunk_base, r=r):
                out = []
                for s in range(2):
                    m1, i1, m2, i2 = c[4 * s:4 * s + 4]
                    off = i * (2 * _LANES) + s * _LANES
                    x = xb[r, pl.ds(off, _LANES)]
                    g = gb[r, pl.ds(off, _LANES)]
                    v = x + t * g
                    cur = lane + (chunk_base + off)
                    gt1 = v > m1
                    loser = jnp.minimum(v, m1)
                    loser_i = jnp.where(gt1, i1, cur)
                    gt2 = loser > m2
                    m2n = jnp.where(gt2, loser, m2)
                    i2n = jnp.where(gt2, loser_i, i2)
                    m1n = jnp.maximum(v, m1)
                    i1n = jnp.where(gt1, cur, i1)
                    out += [m1n, i1n, m2n, i2n]
                return tuple(out)

            carry = tuple(
                ref[r, pl.ds(s * _LANES, _LANES)]
                for s in range(2) for ref in (sm1, si1, sm2, si2))
            carry = lax.fori_loop(0, nvec // 2, body, carry, unroll=4)
            for s in range(2):
                sm1[r, pl.ds(s * _LANES, _LANES)] = carry[4 * s + 0]
                si1[r, pl.ds(s * _LANES, _LANES)] = carry[4 * s + 1]
                sm2[r, pl.ds(s * _LANES, _LANES)] = carry[4 * s + 2]
                si2[r, pl.ds(s * _LANES, _LANES)] = carry[4 * s + 3]

    issue(0, 0)
    issue(1, 1)

    def chunk_pair(p, _):
        k = p * 2
        for b in range(2):
            cur = k + b
            drain(b)
            scan_buf(xbufs.at[b], gbufs.at[b], _C // _LANES, base + cur * _C)

            @pl.when(cur + 2 < _NCH)
            def _(cur=cur, b=b):
                issue(cur + 2, b)
        return _

    lax.fori_loop(0, _NCH // 2, chunk_pair, None)

    pltpu.sync_copy(logits_hbm.at[pl.ds(row0, _RPG), pl.ds(_TAIL, _TAILN)], xtail)
    pltpu.sync_copy(gumbel_hbm.at[pl.ds(row0, _RPG), pl.ds(_TAIL, _TAILN)], gtail)
    scan_buf(xtail, gtail, _TAILN // _LANES, _TAIL)

    out0 = pl.multiple_of(half * _ROWS + grp * _RPG, 8)
    pltpu.sync_copy(sm1, m1_hbm.at[pl.ds(out0, _RPG)])
    pltpu.sync_copy(si1, i1_hbm.at[pl.ds(out0, _RPG)])
    pltpu.sync_copy(sm2, m2_hbm.at[pl.ds(out0, _RPG)])
    pltpu.sync_copy(si2, i2_hbm.at[pl.ds(out0, _RPG)])


_sc_top2 = functools.partial(
    pl.kernel,
    mesh=plsc.VectorSubcoreMesh(core_axis_name="c", subcore_axis_name="s"),
    out_type=[
        jax.ShapeDtypeStruct((2 * _ROWS, 2 * _LANES), jnp.float32),
        jax.ShapeDtypeStruct((2 * _ROWS, 2 * _LANES), jnp.int32),
        jax.ShapeDtypeStruct((2 * _ROWS, 2 * _LANES), jnp.float32),
        jax.ShapeDtypeStruct((2 * _ROWS, 2 * _LANES), jnp.int32),
    ],
    scratch_types=[
        pltpu.VMEM((2, _RPG, _C), jnp.float32),
        pltpu.VMEM((2, _RPG, _C), jnp.float32),
        pltpu.VMEM((_RPG, _TAILN), jnp.float32),
        pltpu.VMEM((_RPG, _TAILN), jnp.float32),
        pltpu.VMEM((_RPG, _LANES), jnp.float32),
        pltpu.VMEM((_RPG, 2 * _LANES), jnp.float32),
        pltpu.VMEM((_RPG, 2 * _LANES), jnp.int32),
        pltpu.VMEM((_RPG, 2 * _LANES), jnp.float32),
        pltpu.VMEM((_RPG, 2 * _LANES), jnp.int32),
        pltpu.SemaphoreType.DMA,
        pltpu.SemaphoreType.DMA,
    ],
)(_sc_body)


def _tc_top2_kernel(logits_ref, gumbel_ref, temp_ref, i1_ref, i2_ref):
    x = logits_ref[...]
    t = temp_ref[...]
    v = x + t * gumbel_ref[...]
    i1 = jnp.argmax(v, axis=-1)
    iota = jax.lax.broadcasted_iota(jnp.int32, v.shape, 1)
    v2 = jnp.where(iota == i1[:, None], -jnp.inf, v)
    i2 = jnp.argmax(v2, axis=-1)
    i1_ref[...] = i1[:, None]
    i2_ref[...] = i2[:, None]


def _tc_top2(logits, gumbel, temps):
    return pl.pallas_call(
        _tc_top2_kernel,
        grid=(_ROWS // 16,),
        in_specs=[
            pl.BlockSpec((16, _W), lambda i: (i, 0)),
            pl.BlockSpec((16, _W), lambda i: (i, 0)),
            pl.BlockSpec((16, 1), lambda i: (i, 0)),
        ],
        out_specs=[
            pl.BlockSpec((16, 1), lambda i: (i, 0)),
            pl.BlockSpec((16, 1), lambda i: (i, 0)),
        ],
        out_shape=[
            jax.ShapeDtypeStruct((_ROWS, 1), jnp.int32),
            jax.ShapeDtypeStruct((_ROWS, 1), jnp.int32),
        ],
    )(logits, gumbel, temps)


def kernel(logits, temperatures):
    logits = logits.astype(jnp.float32)
    temps = temperatures.astype(jnp.float32)
    gumbel = jnp.asarray(_GUMBEL)
    tb = jnp.broadcast_to(temps[:, None], (_ROWS, _LANES))
    m1, i1, m2, i2 = _sc_top2(logits, gumbel, tb)
    t1, t2 = _tc_top2(logits, gumbel, temps.reshape(_ROWS, 1))

    vals = jnp.concatenate(
        [m1[:_ROWS], m1[_ROWS:], m2[:_ROWS], m2[_ROWS:]], axis=1)
    idxs = jnp.concatenate(
        [i1[:_ROWS], i1[_ROWS:], i2[:_ROWS], i2[_ROWS:]], axis=1)
    a1 = jnp.argmax(vals, axis=1)
    c1 = jnp.take_along_axis(idxs, a1[:, None], axis=1)[:, 0]
    vals2 = jnp.where(idxs == c1[:, None], -jnp.inf, vals)
    a2 = jnp.argmax(vals2, axis=1)
    c2 = jnp.take_along_axis(idxs, a2[:, None], axis=1)[:, 0]
    cand = jnp.concatenate(
        [t1, t2, c1[:, None], c2[:, None]], axis=1)

    xg = jnp.take_along_axis(logits, cand, axis=1)
    ng = jnp.take_along_axis(jnp.asarray(_INV_NOISE), cand, axis=1)
    safe_t = jnp.maximum(temps[:, None], 1e-10)
    w = xg / safe_t
    r = jnp.exp(w - jnp.max(w, axis=1, keepdims=True)) * ng
    pick = jnp.argmax(r, axis=1)
    sample = jnp.take_along_axis(cand, pick[:, None], axis=1)[:, 0]

    g_hi = jnp.where(
        xg[:, 0] > xg[:, 1],
        cand[:, 0],
        jnp.where(xg[:, 1] > xg[:, 0], cand[:, 1],
                  jnp.minimum(cand[:, 0], cand[:, 1])),
    )
    return jnp.where(temps <= 1e-10, g_hi, sample)
